# Initial kernel scaffold; baseline (speedup 1.0000x reference)
import jax
import jax.numpy as jnp
from jax.experimental import pallas as pl


def kernel(x, w1, w2):
    raise NotImplementedError("write your pallas kernel here")


# speedup vs baseline: 3.5468x; 3.5468x over previous
import functools

import jax
import jax.numpy as jnp
from jax import lax
from jax.experimental import pallas as pl
from jax.experimental.pallas import tpu as pltpu
from jax.experimental.pallas import tpu_sc as plsc

_NC = 2
_NS = 16
_NW = _NC * _NS


@functools.partial(jax.jit, static_argnames=("chunk",))
def _emb_call(x_flat, w1, w2, *, chunk):
    bf = x_flat.shape[0]
    pivot = w1.shape[0]
    d = w1.shape[1]
    per_w = bf // _NW
    nchunk = per_w // chunk
    assert per_w % chunk == 0 and bf % _NW == 0

    mesh = plsc.VectorSubcoreMesh(
        core_axis_name="c", subcore_axis_name="s",
        num_cores=_NC, num_subcores=_NS,
    )

    @functools.partial(
        pl.kernel,
        out_type=jax.ShapeDtypeStruct((bf, d), jnp.float32),
        mesh=mesh,
        compiler_params=pltpu.CompilerParams(
            needs_layout_passes=False, use_tc_tiling_on_sc=False,
        ),
        scratch_types=[
            pltpu.VMEM((chunk,), jnp.int32),
            pltpu.VMEM((chunk,), jnp.int32),
            pltpu.VMEM((chunk,), jnp.int32),
            pltpu.VMEM((chunk, d), jnp.float32),
            pltpu.VMEM((chunk, d), jnp.float32),
            pltpu.SemaphoreType.DMA,
            pltpu.SemaphoreType.DMA,
        ],
    )
    def emb(x_hbm, w1_hbm, w2_hbm, out_hbm, xv, i1v, i2v, r1v, r2v, sem1, sem2):
        wid = lax.axis_index("s") * _NC + lax.axis_index("c")
        base = wid * per_w

        def chunk_body(j, carry):
            cbase = base + j * chunk
            pltpu.sync_copy(x_hbm.at[pl.ds(cbase, chunk)], xv)

            def prep(g, c):
                xx = xv[pl.ds(g * 16, 16)]
                m = xx < pivot
                i1v[pl.ds(g * 16, 16)] = jnp.where(m, xx, 0)
                i2v[pl.ds(g * 16, 16)] = jnp.where(m, 0, xx - pivot)
                return c

            lax.fori_loop(0, chunk // 16, prep, 0)

            cp1 = pltpu.async_copy(w1_hbm.at[i1v], r1v, sem1)
            cp2 = pltpu.async_copy(w2_hbm.at[i2v], r2v, sem2)
            cp1.wait()
            cp2.wait()

            def merge(row, c):
                mv = plsc.load_gather(xv, [jnp.full((16,), row, jnp.int32)])
                keep1 = mv < pivot
                for v in range(d // 16):
                    a = r1v[row, pl.ds(v * 16, 16)]
                    b = r2v[row, pl.ds(v * 16, 16)]
                    r1v[row, pl.ds(v * 16, 16)] = jnp.where(keep1, a, b)
                return c

            lax.fori_loop(0, chunk, merge, 0)

            pltpu.sync_copy(r1v, out_hbm.at[pl.ds(cbase, chunk)])
            return carry

        lax.fori_loop(0, nchunk, chunk_body, 0)

    return emb(x_flat, w1, w2)


def kernel(x, w1, w2):
    b, f = x.shape
    d = w1.shape[1]
    flat = x.reshape(-1).astype(jnp.int32)
    out = _emb_call(flat, w1, w2, chunk=512)
    return out.reshape(b, f, d)
